# baseline (speedup 1.0000x reference)
import functools

import jax
import jax.numpy as jnp
from jax import lax
from jax.experimental import pallas as pl
from jax.experimental.pallas import tpu as pltpu
from jax.experimental.pallas import tpu_sc as plsc

_NUM_CORES = 2
_NUM_SUBCORES = 16
_NW = _NUM_CORES * _NUM_SUBCORES
_CHUNK = 32
_DEPTH = 3


@functools.lru_cache(maxsize=None)
def _make_gather(v, d, n_chunks, chunk, depth):
    mesh = plsc.VectorSubcoreMesh(core_axis_name="c", subcore_axis_name="s")

    @functools.partial(
        pl.kernel,
        mesh=mesh,
        out_type=jax.ShapeDtypeStruct((_NW * n_chunks * chunk, d), jnp.float32),
        scratch_types=[
            pltpu.VMEM((n_chunks, chunk), jnp.int32),
        ] + [pltpu.VMEM((chunk, d), jnp.float32) for _ in range(depth)]
          + [pltpu.SemaphoreType.DMA for _ in range(2 * depth)],
    )
    def gather_kernel(idx_hbm, table_hbm, out_hbm, idx_v, *bufs):
        rows = bufs[:depth]
        gsem = bufs[depth:2 * depth]
        osem = bufs[2 * depth:]
        wid = lax.axis_index("s") * _NUM_CORES + lax.axis_index("c")
        base = wid * (n_chunks * chunk)
        pltpu.sync_copy(idx_hbm.at[wid], idx_v)

        gathers = {}
        outs = {}
        for j in range(min(depth, n_chunks)):
            gathers[j] = pltpu.async_copy(
                table_hbm.at[idx_v.at[j]], rows[j % depth], gsem[j % depth])
        for j in range(n_chunks):
            b = j % depth
            gathers[j].wait()
            outs[j] = pltpu.async_copy(
                rows[b], out_hbm.at[pl.ds(base + j * chunk, chunk)], osem[b])
            k = j - 1 + depth
            if j >= 1 and k < n_chunks:
                outs[j - 1].wait()
                gathers[k] = pltpu.async_copy(
                    table_hbm.at[idx_v.at[k]], rows[k % depth], gsem[k % depth])
        for j in range(max(0, n_chunks - depth), n_chunks):
            outs[j].wait()

    return gather_kernel


def kernel(x, table):
    b, s = x.shape
    v, d = table.shape
    n = b * s
    n_chunks = n // (_NW * _CHUNK)
    idx = x.reshape(_NW, n_chunks, _CHUNK).astype(jnp.int32)
    rows = _make_gather(v, d, n_chunks, _CHUNK, _DEPTH)(idx, table)
    return rows.reshape(b, s, d)
